# Initial kernel scaffold; baseline (speedup 1.0000x reference)
import jax
import jax.numpy as jnp
from jax.experimental import pallas as pl


def kernel(inputs, theta_w, theta_b, g_w, g_b, random_matrices):
    raise NotImplementedError("write your pallas kernel here")


# speedup vs baseline: 16.9032x; 16.9032x over previous
import functools

import jax
import jax.numpy as jnp
from jax import lax
from jax.experimental import pallas as pl

A_TABLES = 4
K_BUCKET = 144
N_IMG, C_CH, H_IMG, W_IMG = 10, 128, 48, 48
HW = H_IMG * W_IMG
M_PROJ = 16
N_BUCKETS = HW // K_BUCKET
LANE = 128
N_CHUNKS = HW // LANE


def _proj_kernel(x_ref, w_ref, b_ref, hash_ref, uw_ref):
    x = x_ref[0]
    p = lax.dot_general(x, w_ref[...], (((0,), (0,)), ((), ())),
                        preferred_element_type=jnp.float32)
    p = p + b_ref[...]
    for a in range(A_TABLES):
        h = jnp.max(p[:, a * M_PROJ:(a + 1) * M_PROJ], axis=1)
        hash_ref[0, a, :] = h
    m1 = p[:, 64:67]
    m2 = p[:, 67:70]
    uw = jnp.concatenate([m1 * m2, m1, jnp.zeros((HW, 10), jnp.float32)],
                         axis=1)
    uw_ref[0] = uw


def _rank_kernel(hrow_ref, hcol_ref, rank_ref):
    hcol = hcol_ref[0]
    ii = lax.broadcasted_iota(jnp.int32, (HW, LANE), 0)
    jj0 = lax.broadcasted_iota(jnp.int32, (HW, LANE), 1)
    acc = jnp.zeros((HW, LANE), jnp.float32)
    for c in range(N_CHUNKS):
        hj = hrow_ref[0, 0, pl.ds(c * LANE, LANE)].reshape(1, LANE)
        gt = hj > hcol
        eq = hj == hcol
        before = (jj0 + (c * LANE)) < ii
        acc = acc + jnp.where(gt | (eq & before), 1.0, 0.0)
    rank_ref[0] = jnp.sum(acc, axis=1, keepdims=True).astype(jnp.int32)


def _out_kernel(rank_ref, uw_ref, out_ref):
    rank = rank_ref[0]
    bucket = rank // K_BUCKET
    bids = lax.broadcasted_iota(jnp.int32, (1, N_BUCKETS), 1)
    onehot = jnp.where(bucket == bids, 1.0, 0.0)
    uw = uw_ref[0]
    sums = lax.dot_general(onehot, uw, (((0,), (0,)), ((), ())),
                           preferred_element_type=jnp.float32,
                           precision=lax.Precision.HIGHEST)
    val = sums[:, 0:3] / sums[:, 3:6]
    val3 = lax.dot_general(onehot, val, (((1,), (0,)), ((), ())),
                           preferred_element_type=jnp.float32,
                           precision=lax.Precision.HIGHEST)
    full = jnp.concatenate(
        [val3, jnp.zeros((HW, C_CH - 3), jnp.float32)], axis=1)
    out_ref[0, 0] = full


def kernel(inputs, theta_w, theta_b, g_w, g_b, random_matrices):
    N, C, H, W = inputs.shape
    x3 = inputs.reshape(N, C, HW)
    wcat = jnp.concatenate(
        [random_matrices.transpose(1, 0, 2).reshape(C, A_TABLES * M_PROJ),
         theta_w, g_w], axis=1)
    bcat = jnp.concatenate(
        [jnp.zeros((64,), jnp.float32), theta_b, g_b]).reshape(1, 70)

    hash_nah, uw = pl.pallas_call(
        _proj_kernel,
        grid=(N,),
        in_specs=[
            pl.BlockSpec((1, C, HW), lambda n: (n, 0, 0)),
            pl.BlockSpec((C, 70), lambda n: (0, 0)),
            pl.BlockSpec((1, 70), lambda n: (0, 0)),
        ],
        out_specs=[
            pl.BlockSpec((1, A_TABLES, HW), lambda n: (n, 0, 0)),
            pl.BlockSpec((1, HW, 16), lambda n: (n, 0, 0)),
        ],
        out_shape=[
            jax.ShapeDtypeStruct((N, A_TABLES, HW), jnp.float32),
            jax.ShapeDtypeStruct((N, HW, 16), jnp.float32),
        ],
    )(x3, wcat, bcat)

    hrows = hash_nah.reshape(N * A_TABLES, 1, HW)
    hcols = hash_nah.reshape(N * A_TABLES, HW, 1)

    rank_t = pl.pallas_call(
        _rank_kernel,
        grid=(N * A_TABLES,),
        in_specs=[
            pl.BlockSpec((1, 1, HW), lambda r: (r, 0, 0)),
            pl.BlockSpec((1, HW, 1), lambda r: (r, 0, 0)),
        ],
        out_specs=pl.BlockSpec((1, HW, 1), lambda r: (r, 0, 0)),
        out_shape=jax.ShapeDtypeStruct((N * A_TABLES, HW, 1), jnp.int32),
    )(hrows, hcols)

    out = pl.pallas_call(
        _out_kernel,
        grid=(A_TABLES, N),
        in_specs=[
            pl.BlockSpec((1, HW, 1), lambda a, n: (n * A_TABLES + a, 0, 0)),
            pl.BlockSpec((1, HW, 16), lambda a, n: (n, 0, 0)),
        ],
        out_specs=pl.BlockSpec((1, 1, HW, C_CH), lambda a, n: (a, n, 0, 0)),
        out_shape=jax.ShapeDtypeStruct((A_TABLES, N, HW, C_CH), jnp.float32),
    )(rank_t, uw)
    return out


# speedup vs baseline: 21.5845x; 1.2770x over previous
import functools

import jax
import jax.numpy as jnp
from jax import lax
from jax.experimental import pallas as pl
from jax.experimental.pallas import tpu as pltpu
from jax.experimental.pallas import tpu_sc as plsc

A_TABLES = 4
K_BUCKET = 144
N_IMG, C_CH = 10, 128
HW = 48 * 48
M_PROJ = 16
N_BUCKETS = HW // K_BUCKET
LANE = 128
N_CHUNKS = HW // LANE
N_ROWS = N_IMG * A_TABLES
N_WORKERS = 32


def _proj_kernel(x_ref, wh_ref, wuw_ref, b_ref, hash_ref, uw_ref):
    x = x_ref[0]
    ph = lax.dot_general(wh_ref[...], x, (((0,), (0,)), ((), ())),
                         preferred_element_type=jnp.float32)
    for a in range(A_TABLES):
        hash_ref[0, a, :] = jnp.max(ph[a * M_PROJ:(a + 1) * M_PROJ, :], axis=0)
    p = lax.dot_general(wuw_ref[...], x, (((0,), (0,)), ((), ())),
                        preferred_element_type=jnp.float32)
    p = p + b_ref[...]
    m1 = p[0:3, :]
    m2 = p[3:6, :]
    uw = jnp.concatenate([m1 * m2, m1, jnp.zeros((2, HW), jnp.float32)],
                         axis=0)
    uw_ref[0] = uw


def _rank_kernel(hrow_ref, hcol_ref, rank_ref):
    hcol = hcol_ref[0]
    ii = lax.broadcasted_iota(jnp.int32, (HW, LANE), 0)
    jj0 = lax.broadcasted_iota(jnp.int32, (HW, LANE), 1)
    acc = jnp.zeros((HW, LANE), jnp.float32)
    for c in range(N_CHUNKS):
        hj = hrow_ref[0, 0, pl.ds(c * LANE, LANE)].reshape(1, LANE)
        gt = hj > hcol
        eq = hj == hcol
        before = (jj0 + (c * LANE)) < ii
        acc = acc + jnp.where(gt | (eq & before), 1.0, 0.0)
    rank_ref[0] = jnp.sum(acc, axis=1, keepdims=True).astype(jnp.int32)


def _sc_out_body(ranks_hbm, uw_hbm, out_hbm,
                 rk_v, sidx_v, guw_v, buf_v, sem_g, sem_s0, sem_s1):
    cid = lax.axis_index("c")
    sid = lax.axis_index("s")
    wid = sid * 2 + cid
    lanes = lax.broadcasted_iota(jnp.int32, (16,), 0)
    zero16 = jnp.zeros((16,), jnp.float32)
    for bi in range(2):
        for j in range(16):
            for v in range(8):
                buf_v[bi, j, pl.ds(v * 16, 16)] = zero16

    for t in range(2):
        r = wid + t * N_WORKERS

        @pl.when(r < N_ROWS)
        def _():
            n = lax.shift_right_logical(r, 2)
            a = lax.bitwise_and(r, 3)
            ro = a * N_IMG + n
            pltpu.sync_copy(ranks_hbm.at[r], rk_v)
            pltpu.async_copy(uw_hbm.at[n], guw_v, sem_g).wait()

            def inv_body(i, carry):
                for u in range(8):
                    o = i * LANE + u * 16
                    rr = rk_v[pl.ds(o, 16)]
                    plsc.store_scatter(sidx_v, [rr], lanes + o)
                return carry
            lax.fori_loop(0, N_CHUNKS, inv_body, 0)

            prev = [None, None]
            sems = [sem_s0, sem_s1]
            for b in range(N_BUCKETS):
                def sum_body(i, accs):
                    o = b * K_BUCKET + i * 16
                    rows = sidx_v[pl.ds(o, 16)]
                    return tuple(
                        accs[c] + plsc.load_gather(
                            guw_v, [jnp.full((16,), c, jnp.int32), rows])
                        for c in range(6))
                accs = lax.fori_loop(
                    0, K_BUCKET // 16, sum_body,
                    tuple(jnp.zeros((16,), jnp.float32) for _ in range(6)))
                tots = [jnp.sum(accs[c], axis=0) for c in range(6)]
                num = jnp.where(
                    lanes == 0, tots[0],
                    jnp.where(lanes == 1, tots[1],
                              jnp.where(lanes == 2, tots[2], 0.0)))
                den = jnp.where(
                    lanes == 0, tots[3],
                    jnp.where(lanes == 1, tots[4],
                              jnp.where(lanes == 2, tots[5], 1.0)))
                val = num / den

                bi = b % 2
                if prev[bi] is not None:
                    for h in prev[bi]:
                        h.wait()
                for j in range(16):
                    buf_v[bi, j, pl.ds(0, 16)] = val
                handles = []
                for tc in range(K_BUCKET // 16):
                    o = b * K_BUCKET + tc * 16
                    iv = sidx_v[pl.ds(o, 16)]
                    handles.append(pltpu.async_copy(
                        buf_v.at[bi], out_hbm.at[ro].at[iv], sems[bi]))
                prev[bi] = handles
            for bi in range(2):
                if prev[bi] is not None:
                    for h in prev[bi]:
                        h.wait()


def kernel(inputs, theta_w, theta_b, g_w, g_b, random_matrices):
    N, C, H, W = inputs.shape
    x3 = inputs.reshape(N, C, HW)
    wh = random_matrices.transpose(1, 0, 2).reshape(C, A_TABLES * M_PROJ)
    wuw = jnp.concatenate([theta_w, g_w], axis=1)
    bcat = jnp.concatenate([theta_b, g_b]).reshape(6, 1)

    hash_nah, uw = pl.pallas_call(
        _proj_kernel,
        grid=(N,),
        in_specs=[
            pl.BlockSpec((1, C, HW), lambda n: (n, 0, 0)),
            pl.BlockSpec((C, A_TABLES * M_PROJ), lambda n: (0, 0)),
            pl.BlockSpec((C, 6), lambda n: (0, 0)),
            pl.BlockSpec((6, 1), lambda n: (0, 0)),
        ],
        out_specs=[
            pl.BlockSpec((1, A_TABLES, HW), lambda n: (n, 0, 0)),
            pl.BlockSpec((1, 8, HW), lambda n: (n, 0, 0)),
        ],
        out_shape=[
            jax.ShapeDtypeStruct((N, A_TABLES, HW), jnp.float32),
            jax.ShapeDtypeStruct((N, 8, HW), jnp.float32),
        ],
    )(x3, wh, wuw, bcat)

    hrows = hash_nah.reshape(N * A_TABLES, 1, HW)
    hcols = hash_nah.reshape(N * A_TABLES, HW, 1)

    rank_t = pl.pallas_call(
        _rank_kernel,
        grid=(N * A_TABLES,),
        in_specs=[
            pl.BlockSpec((1, 1, HW), lambda r: (r, 0, 0)),
            pl.BlockSpec((1, HW, 1), lambda r: (r, 0, 0)),
        ],
        out_specs=pl.BlockSpec((1, HW, 1), lambda r: (r, 0, 0)),
        out_shape=jax.ShapeDtypeStruct((N * A_TABLES, HW, 1), jnp.int32),
    )(hrows, hcols)
    ranks = rank_t.reshape(N * A_TABLES, HW)

    sc_out = functools.partial(
        pl.kernel,
        mesh=plsc.VectorSubcoreMesh(core_axis_name="c", subcore_axis_name="s"),
        compiler_params=pltpu.CompilerParams(needs_layout_passes=False),
        out_type=jax.ShapeDtypeStruct((A_TABLES * N_IMG, HW, C_CH),
                                      jnp.float32),
        scratch_types=[
            pltpu.VMEM((HW,), jnp.int32),
            pltpu.VMEM((HW,), jnp.int32),
            pltpu.VMEM((8, HW), jnp.float32),
            pltpu.VMEM((2, 16, C_CH), jnp.float32),
            pltpu.SemaphoreType.DMA,
            pltpu.SemaphoreType.DMA,
            pltpu.SemaphoreType.DMA,
        ],
    )(_sc_out_body)
    out2 = sc_out(ranks, uw)
    return out2.reshape(A_TABLES, N_IMG, HW, C_CH)
